# baseline (speedup 1.0000x reference)
import jax
import jax.numpy as jnp
from jax.experimental import pallas as pl

_T = 32
_BL = 3
_P = 8
_MB = 4
_BB = 8


def _burst_body(x_ref, out_ref):
    v = x_ref[...] * jnp.float32(_MB)
    f = v.shape[-1]
    t = jax.lax.broadcasted_iota(jnp.int32, (_T, f), 0)
    thr = ((t // _P) + 1).astype(jnp.float32)
    within = (t % _P) < _BL
    for b in range(v.shape[0]):
        act = jnp.broadcast_to(v[b:b + 1, :], (_T, f)) >= thr
        out_ref[b, :, :] = (within & act).astype(jnp.float32)


def kernel(x):
    bsz, f = x.shape
    grid = (bsz // _BB,)
    return pl.pallas_call(
        _burst_body,
        grid=grid,
        in_specs=[pl.BlockSpec((_BB, f), lambda i: (i, 0))],
        out_specs=pl.BlockSpec((_BB, _T, f), lambda i: (i, 0, 0)),
        out_shape=jax.ShapeDtypeStruct((bsz, _T, f), jnp.float32),
    )(x)


# speedup vs baseline: 1.1162x; 1.1162x over previous
import jax
import jax.numpy as jnp
from jax.experimental import pallas as pl

_T = 32
_BL = 3
_P = 8
_MB = 4
_BB = 16


def _burst_body(x_ref, out_ref):
    v = x_ref[...] * jnp.float32(_MB)
    f = v.shape[-1]
    t = jax.lax.broadcasted_iota(jnp.int32, (_T, f), 0)
    thr = ((t // _P) + 1).astype(jnp.float32)
    within = (t % _P) < _BL
    for b in range(v.shape[0]):
        act = jnp.broadcast_to(v[b:b + 1, :], (_T, f)) >= thr
        out_ref[b, :, :] = (within & act).astype(jnp.float32)


def kernel(x):
    bsz, f = x.shape
    grid = (bsz // _BB,)
    return pl.pallas_call(
        _burst_body,
        grid=grid,
        in_specs=[pl.BlockSpec((_BB, f), lambda i: (i, 0))],
        out_specs=pl.BlockSpec((_BB, _T, f), lambda i: (i, 0, 0)),
        out_shape=jax.ShapeDtypeStruct((bsz, _T, f), jnp.float32),
    )(x)
